# Initial kernel scaffold; baseline (speedup 1.0000x reference)
import jax
import jax.numpy as jnp
from jax.experimental import pallas as pl


def kernel(row_features, edge_index, edge_attr, variable_features, candidates, params):
    raise NotImplementedError("write your pallas kernel here")


# speedup vs baseline: 2.1122x; 2.1122x over previous
import functools

import jax
import jax.numpy as jnp
from jax import lax
from jax.experimental import pallas as pl
from jax.experimental.pallas import tpu as pltpu
from jax.experimental.pallas import tpu_sc as plsc

NN = 50000
NE = 800000
NCAND = 5000
EMB = 64
HALF = 32
ECH = 128
NCHUNKS = NE // ECH
CCH = 40
NCCH = NCAND // CCH
ZCH = 1000
NZCH = NN // ZCH

f32 = jnp.float32


def _sub_chunk_loop(sid, nch, body):
    def wrapped(i, carry):
        body(sid + i * 16)
        return carry

    n_i = nch // 16 + jnp.where(sid < (nch % 16), 1, 0)
    lax.fori_loop(0, n_i, wrapped, 0)


def _indicator_sc_body(cand, zeros8, ones8, out, idx_v, ones_v, tab_sh):
    cid = lax.axis_index("c")
    sid = lax.axis_index("s")

    def zero(c):
        pltpu.sync_copy(zeros8.at[pl.ds(c * ZCH, ZCH)], tab_sh.at[pl.ds(c * ZCH, ZCH)])

    _sub_chunk_loop(sid, NZCH, zero)
    pltpu.sync_copy(ones8.at[pl.ds(0, CCH)], ones_v)
    plsc.subcore_barrier()

    def scat(c):
        base = c * CCH
        pltpu.sync_copy(cand.at[pl.ds(base, CCH)], idx_v)
        pltpu.sync_copy(ones_v, tab_sh.at[idx_v])

    _sub_chunk_loop(sid, NCCH, scat)
    plsc.subcore_barrier()

    @pl.when(cid == 0)
    def _():
        def wout(c):
            pltpu.sync_copy(tab_sh.at[pl.ds(c * ZCH, ZCH)], out.at[pl.ds(c * ZCH, ZCH)])

        _sub_chunk_loop(sid, NZCH, wout)


def _gather_sc_body(hl, hr, src, dst, ga, gb, sidx_v, didx_v, bufa, bufb, sema, semb):
    cid = lax.axis_index("c")
    sid = lax.axis_index("s")
    wid = sid * 2 + cid

    def body(i, carry):
        base = (wid + i * 32) * ECH
        pltpu.sync_copy(src.at[pl.ds(base, ECH)], sidx_v)
        pltpu.sync_copy(dst.at[pl.ds(base, ECH)], didx_v)
        ca = pltpu.async_copy(hl.at[sidx_v], bufa, sema)
        cb = pltpu.async_copy(hr.at[didx_v], bufb, semb)
        ca.wait()
        cb.wait()
        pltpu.sync_copy(bufa, ga.at[pl.ds(base, ECH)])
        pltpu.sync_copy(bufb, gb.at[pl.ds(base, ECH)])
        return carry

    n_i = NCHUNKS // 32 + jnp.where(wid < (NCHUNKS % 32), 1, 0)
    lax.fori_loop(0, n_i, body, 0)


def _scatter_sc_body(t0, t1, dst, zeros32, zeros8, ones8, sums0, sums1, cnt,
                     didx_v, rows_v, ones_v, tab_sh, cnt_sh):
    cid = lax.axis_index("c")
    sid = lax.axis_index("s")

    def zero(c):
        pltpu.sync_copy(zeros32.at[pl.ds(c * ZCH, ZCH)], tab_sh.at[pl.ds(c * ZCH, ZCH)])
        pltpu.sync_copy(zeros8.at[pl.ds(c * ZCH, ZCH)], cnt_sh.at[pl.ds(c * ZCH, ZCH)])

    _sub_chunk_loop(sid, NZCH, zero)
    pltpu.sync_copy(ones8.at[pl.ds(0, ECH)], ones_v)
    plsc.subcore_barrier()

    def scat(c):
        base = c * ECH
        pltpu.sync_copy(dst.at[pl.ds(base, ECH)], didx_v)

        @pl.when(cid == 0)
        def _():
            pltpu.sync_copy(t0.at[pl.ds(base, ECH)], rows_v)

        @pl.when(cid == 1)
        def _():
            pltpu.sync_copy(t1.at[pl.ds(base, ECH)], rows_v)

        pltpu.sync_copy(rows_v, tab_sh.at[didx_v], add=True)
        pltpu.sync_copy(ones_v, cnt_sh.at[didx_v], add=True)

    _sub_chunk_loop(sid, NCHUNKS, scat)
    plsc.subcore_barrier()

    @pl.when(cid == 0)
    def _():
        def wout(c):
            pltpu.sync_copy(tab_sh.at[pl.ds(c * ZCH, ZCH)], sums0.at[pl.ds(c * ZCH, ZCH)])
            pltpu.sync_copy(cnt_sh.at[pl.ds(c * ZCH, ZCH)], cnt.at[pl.ds(c * ZCH, ZCH)])

        _sub_chunk_loop(sid, NZCH, wout)

    @pl.when(cid == 1)
    def _():
        def wout(c):
            pltpu.sync_copy(tab_sh.at[pl.ds(c * ZCH, ZCH)], sums1.at[pl.ds(c * ZCH, ZCH)])

        _sub_chunk_loop(sid, NZCH, wout)


def _cand_gather_sc_body(table, cand, out, idx_v, rows_v, sem):
    cid = lax.axis_index("c")
    sid = lax.axis_index("s")
    wid = sid * 2 + cid

    def body(i, carry):
        base = (wid + i * 32) * CCH
        pltpu.sync_copy(cand.at[pl.ds(base, CCH)], idx_v)
        pltpu.async_copy(table.at[idx_v], rows_v, sem).wait()
        pltpu.sync_copy(rows_v, out.at[pl.ds(base, CCH)])
        return carry

    n_i = NCCH // 32 + jnp.where(wid < (NCCH % 32), 1, 0)
    lax.fori_loop(0, n_i, body, 0)


@functools.cache
def _sc_kernels():
    mesh = plsc.VectorSubcoreMesh(core_axis_name="c", subcore_axis_name="s",
                                  num_cores=2, num_subcores=16)
    params = pltpu.CompilerParams(use_tc_tiling_on_sc=False)
    indicator = pl.kernel(
        _indicator_sc_body,
        out_type=jax.ShapeDtypeStruct((NN, 8), f32),
        mesh=mesh,
        compiler_params=params,
        scratch_types=[
            pltpu.VMEM((CCH,), jnp.int32),
            pltpu.VMEM((CCH, 8), f32),
            pltpu.VMEM_SHARED((NN, 8), f32),
        ],
    )
    gather = pl.kernel(
        _gather_sc_body,
        out_type=(
            jax.ShapeDtypeStruct((NE, EMB), f32),
            jax.ShapeDtypeStruct((NE, EMB), f32),
        ),
        mesh=mesh,
        compiler_params=params,
        scratch_types=[
            pltpu.VMEM((ECH,), jnp.int32),
            pltpu.VMEM((ECH,), jnp.int32),
            pltpu.VMEM((ECH, EMB), f32),
            pltpu.VMEM((ECH, EMB), f32),
            pltpu.SemaphoreType.DMA,
            pltpu.SemaphoreType.DMA,
        ],
    )
    scatter = pl.kernel(
        _scatter_sc_body,
        out_type=(
            jax.ShapeDtypeStruct((NN, HALF), f32),
            jax.ShapeDtypeStruct((NN, HALF), f32),
            jax.ShapeDtypeStruct((NN, 8), f32),
        ),
        mesh=mesh,
        compiler_params=params,
        scratch_types=[
            pltpu.VMEM((ECH,), jnp.int32),
            pltpu.VMEM((ECH, HALF), f32),
            pltpu.VMEM((ECH, 8), f32),
            pltpu.VMEM_SHARED((NN, HALF), f32),
            pltpu.VMEM_SHARED((NN, 8), f32),
        ],
    )
    cand_gather = pl.kernel(
        _cand_gather_sc_body,
        out_type=jax.ShapeDtypeStruct((NCAND, 8), f32),
        mesh=mesh,
        compiler_params=params,
        scratch_types=[
            pltpu.VMEM((CCH,), jnp.int32),
            pltpu.VMEM((CCH, 8), f32),
            pltpu.SemaphoreType.DMA,
        ],
    )
    return indicator, gather, scatter, cand_gather


def _ln(x, g, b):
    m = jnp.mean(x, axis=-1, keepdims=True)
    v = jnp.mean((x - m) ** 2, axis=-1, keepdims=True)
    return (x - m) * lax.rsqrt(v + 1e-5) * g + b


def _lrelu(x):
    return jnp.where(x > 0, x, 0.01 * x)


def _mm(x, w):
    return lax.dot_general(x, w, (((1,), (1,)), ((), ())),
                           preferred_element_type=f32)


RB = 2000
RBE = 2000


def _row_spec(width):
    return pl.BlockSpec((RB, width), lambda i: (i, 0))


def _full_spec(shape):
    nd = len(shape)
    return pl.BlockSpec(shape, lambda i: (0,) * nd)


def _bc8(v):
    return jnp.broadcast_to(v.reshape(1, -1), (8, v.shape[-1]))


def _embed_body(rf, vf, ind,
                g0c, b0c, w1c, b1c, w2c, b2c,
                g0v, b0v, w1v, b1v, w2v, b2v,
                pe0, ped, wl, bl, wr,
                cons_o, var_o, hl_o, hr_o):
    h = _ln(rf[...], g0c[...][0:1, :], b0c[...][0:1, :])
    h = _lrelu(_mm(h, w1c[...]) + b1c[...][0:1, :])
    h = _lrelu(_mm(h, w2c[...]) + b2c[...][0:1, :])
    cons_o[...] = h
    hr_o[...] = _mm(h, wr[...])

    v = _ln(vf[...], g0v[...][0:1, :], b0v[...][0:1, :])
    v = _lrelu(_mm(v, w1v[...]) + b1v[...][0:1, :])
    v = _lrelu(_mm(v, w2v[...]) + b2v[...][0:1, :])
    v = v + pe0[...][0:1, :] + ind[...][:, 0:1] * ped[...][0:1, :]
    var_o[...] = v
    hl_o[...] = _mm(v, wl[...]) + bl[...][0:1, :]


def _embed_tc(rf, vf, ind8, ce, ve, pe0, ped, wl, bl, wr):
    outs = tuple(jax.ShapeDtypeStruct((NN, EMB), f32) for _ in range(4))
    w_specs = [
        _full_spec((8, 5)), _full_spec((8, 5)),
        _full_spec((EMB, 5)), _full_spec((8, EMB)),
        _full_spec((EMB, EMB)), _full_spec((8, EMB)),
        _full_spec((8, 19)), _full_spec((8, 19)),
        _full_spec((EMB, 19)), _full_spec((8, EMB)),
        _full_spec((EMB, EMB)), _full_spec((8, EMB)),
        _full_spec((8, EMB)), _full_spec((8, EMB)),
        _full_spec((EMB, EMB)), _full_spec((8, EMB)),
        _full_spec((EMB, EMB)),
    ]
    return pl.pallas_call(
        _embed_body,
        grid=(NN // RB,),
        in_specs=[_row_spec(5), _row_spec(19), _row_spec(8)] + w_specs,
        out_specs=tuple(_row_spec(EMB) for _ in range(4)),
        out_shape=outs,
    )(rf, vf, ind8,
      _bc8(ce['g0']), _bc8(ce['b0']), ce['W1'], _bc8(ce['b1']), ce['W2'], _bc8(ce['b2']),
      _bc8(ve['g0']), _bc8(ve['b0']), ve['W1'], _bc8(ve['b1']), ve['W2'], _bc8(ve['b2']),
      pe0, ped, wl, _bc8(bl), wr)


def _edge_body(ga, gb, g, b, t0_o, t1_o):
    t = _lrelu(_ln(ga[...] + gb[...], g[...][0:1, :], b[...][0:1, :]))
    t0_o[...] = t[:, :HALF]
    t1_o[...] = t[:, HALF:]


def _edge_tc(ga, gb, g, b):
    spec64 = pl.BlockSpec((RBE, EMB), lambda i: (i, 0))
    spec32 = pl.BlockSpec((RBE, HALF), lambda i: (i, 0))
    return pl.pallas_call(
        _edge_body,
        grid=(NE // RBE,),
        in_specs=[spec64, spec64, _full_spec((8, EMB)), _full_spec((8, EMB))],
        out_specs=(spec32, spec32),
        out_shape=(jax.ShapeDtypeStruct((NE, HALF), f32),
                   jax.ShapeDtypeStruct((NE, HALF), f32)),
    )(ga, gb, _bc8(g), _bc8(b))


def _post1_body(s0, s1, cnt, cons, var,
                wf, bf, g, b, wo1, bo1, wo2, bo2, wl2, bl2, wr2,
                hl2_o, hr2_o):
    sums = jnp.concatenate([s0[...], s1[...]], axis=-1)
    c = cnt[...][:, 0:1]
    mean = sums / jnp.maximum(c, 1.0)
    gate = (c > 0).astype(f32)
    agg = _mm(mean, wf[...]) + bf[...][0:1, :] * gate
    h = _ln(agg, g[...][0:1, :], b[...][0:1, :])
    hcat = jnp.concatenate([h, cons[...]], axis=-1)
    ho = _lrelu(_mm(hcat, wo1[...]) + bo1[...][0:1, :])
    consn = _mm(ho, wo2[...]) + bo2[...][0:1, :]
    hl2_o[...] = _mm(consn, wl2[...]) + bl2[...][0:1, :]
    hr2_o[...] = _mm(var[...], wr2[...])


def _post1_tc(s0, s1, cnt, cons, var, p1, wl2, bl2, wr2):
    w_specs = [
        _full_spec((EMB, EMB)), _full_spec((8, EMB)),
        _full_spec((8, EMB)), _full_spec((8, EMB)),
        _full_spec((EMB, 2 * EMB)), _full_spec((8, EMB)),
        _full_spec((EMB, EMB)), _full_spec((8, EMB)),
        _full_spec((EMB, EMB)), _full_spec((8, EMB)),
        _full_spec((EMB, EMB)),
    ]
    return pl.pallas_call(
        _post1_body,
        grid=(NN // RB,),
        in_specs=[_row_spec(HALF), _row_spec(HALF), _row_spec(8),
                  _row_spec(EMB), _row_spec(EMB)] + w_specs,
        out_specs=(_row_spec(EMB), _row_spec(EMB)),
        out_shape=(jax.ShapeDtypeStruct((NN, EMB), f32),
                   jax.ShapeDtypeStruct((NN, EMB), f32)),
    )(s0, s1, cnt, cons, var,
      p1['Wf'], _bc8(p1['bf']), _bc8(p1['ln_pc_g']), _bc8(p1['ln_pc_b']),
      p1['Wo1'], _bc8(p1['bo1']), p1['Wo2'], _bc8(p1['bo2']),
      wl2, _bc8(bl2), wr2)


def _post2_body(s0, s1, cnt, var,
                wf, bf, g, b, wo1, bo1, wo2, bo2, w1h, b1h, w2h, b2h,
                out_o):
    sums = jnp.concatenate([s0[...], s1[...]], axis=-1)
    c = cnt[...][:, 0:1]
    mean = sums / jnp.maximum(c, 1.0)
    gate = (c > 0).astype(f32)
    agg = _mm(mean, wf[...]) + bf[...][0:1, :] * gate
    h = _ln(agg, g[...][0:1, :], b[...][0:1, :])
    hcat = jnp.concatenate([h, var[...]], axis=-1)
    ho = _lrelu(_mm(hcat, wo1[...]) + bo1[...][0:1, :])
    varn = _mm(ho, wo2[...]) + bo2[...][0:1, :]
    hh = _lrelu(_mm(varn, w1h[...]) + b1h[...][0:1, :])
    out_o[...] = _mm(hh, w2h[...]) + b2h[...][0:1, :]


def _post2_tc(s0, s1, cnt, var, p2, head):
    w_specs = [
        _full_spec((EMB, EMB)), _full_spec((8, EMB)),
        _full_spec((8, EMB)), _full_spec((8, EMB)),
        _full_spec((EMB, 2 * EMB)), _full_spec((8, EMB)),
        _full_spec((EMB, EMB)), _full_spec((8, EMB)),
        _full_spec((EMB, EMB)), _full_spec((8, EMB)),
        _full_spec((8, EMB)), _full_spec((8, 8)),
    ]
    b2h = jnp.broadcast_to(head['b2'].reshape(1, 1), (8, 8))
    return pl.pallas_call(
        _post2_body,
        grid=(NN // RB,),
        in_specs=[_row_spec(HALF), _row_spec(HALF), _row_spec(8),
                  _row_spec(EMB)] + w_specs,
        out_specs=_row_spec(8),
        out_shape=jax.ShapeDtypeStruct((NN, 8), f32),
    )(s0, s1, cnt, var,
      p2['Wf'], _bc8(p2['bf']), _bc8(p2['ln_pc_g']), _bc8(p2['ln_pc_b']),
      p2['Wo1'], _bc8(p2['bo1']), p2['Wo2'], _bc8(p2['bo2']),
      head['W1'], _bc8(head['b1']), _bc8(head['W2'][0]), b2h)


def kernel(row_features, edge_index, edge_attr, variable_features, candidates, params):
    del edge_attr
    e0 = edge_index[0].astype(jnp.int32)
    e1 = edge_index[1].astype(jnp.int32)
    cand = candidates.astype(jnp.int32)

    zeros32 = jnp.zeros((NN, HALF), f32)
    zeros8 = jnp.zeros((NN, 8), f32)
    ones8 = jnp.ones((ECH, 8), f32)

    pe = params['pos_emb']
    pe0 = _bc8(pe[0])
    ped = _bc8(pe[1] - pe[0])

    p1 = params['conv_vc']
    p2 = params['conv_cv']

    _indicator_sc, _gather_sc, _scatter_sc, _cand_gather_sc = _sc_kernels()

    ind8 = _indicator_sc(cand, zeros8, ones8)
    cons, var, hl1, hr1 = _embed_tc(
        row_features, variable_features, ind8,
        params['cons_emb'], params['var_emb'], pe0, ped,
        p1['Wl'], p1['bl'], p1['Wr'])

    ga, gb = _gather_sc(hl1, hr1, e1, e0)
    t0, t1 = _edge_tc(ga, gb, p1['ln_f_g'], p1['ln_f_b'])
    s0, s1, cnt_c = _scatter_sc(t0, t1, e0, zeros32, zeros8, ones8)
    hl2, hr2 = _post1_tc(s0, s1, cnt_c, cons, var, p1,
                         p2['Wl'], p2['bl'], p2['Wr'])

    ga2, gb2 = _gather_sc(hl2, hr2, e0, e1)
    t0b, t1b = _edge_tc(ga2, gb2, p2['ln_f_g'], p2['ln_f_b'])
    s0b, s1b, cnt_v = _scatter_sc(t0b, t1b, e1, zeros32, zeros8, ones8)
    out2d = _post2_tc(s0b, s1b, cnt_v, var, p2, params['head'])

    res2d = _cand_gather_sc(out2d, cand)
    return res2d[:, 0]


# speedup vs baseline: 2.5214x; 1.1938x over previous
import functools

import jax
import jax.numpy as jnp
from jax import lax
from jax.experimental import pallas as pl
from jax.experimental.pallas import tpu as pltpu
from jax.experimental.pallas import tpu_sc as plsc

NN = 50000
NE = 800000
NCAND = 5000
EMB = 64
HALF = 32
ECH = 128
NCHUNKS = NE // ECH
CCH = 40
NCCH = NCAND // CCH
ZCH = 1000
NZCH = NN // ZCH

f32 = jnp.float32


def _sub_chunk_loop(sid, nch, body):
    def wrapped(i, carry):
        body(sid + i * 16)
        return carry

    n_i = nch // 16 + jnp.where(sid < (nch % 16), 1, 0)
    lax.fori_loop(0, n_i, wrapped, 0)


def _stats_sc_body(cand, e0, e1, zeros8, ones8, ind_o, cntc_o, cntv_o,
                   cidx_v, didx0, didx1, ones_v,
                   semi0, semi1, sems0, sems1,
                   ind_sh, cnt_sh):
    cid = lax.axis_index("c")
    sid = lax.axis_index("s")

    def zero(c):
        pltpu.sync_copy(zeros8.at[pl.ds(c * ZCH, ZCH)], ind_sh.at[pl.ds(c * ZCH, ZCH)])
        pltpu.sync_copy(zeros8.at[pl.ds(c * ZCH, ZCH)], cnt_sh.at[pl.ds(c * ZCH, ZCH)])

    _sub_chunk_loop(sid, NZCH, zero)
    pltpu.sync_copy(ones8.at[pl.ds(0, ECH)], ones_v)
    plsc.subcore_barrier()

    def scat(c):
        base = c * CCH
        pltpu.sync_copy(cand.at[pl.ds(base, CCH)], cidx_v)
        pltpu.sync_copy(ones_v.at[pl.ds(0, CCH)], ind_sh.at[cidx_v])

    _sub_chunk_loop(sid, NCCH, scat)

    didx = (didx0, didx1)
    semi = (semi0, semi1)
    sems = (sems0, sems1)
    n_i = NCHUNKS // 16 + jnp.where(sid < (NCHUNKS % 16), 1, 0)

    def base(k):
        return (sid + k * 16) * ECH

    def start_fetch(k, b):
        @pl.when(cid == 0)
        def _():
            pltpu.async_copy(e0.at[pl.ds(base(k), ECH)], didx[b], semi[b])

        @pl.when(cid == 1)
        def _():
            pltpu.async_copy(e1.at[pl.ds(base(k), ECH)], didx[b], semi[b])

    def wait_fetch(b):
        pltpu.make_async_copy(e0.at[pl.ds(0, ECH)], didx[b], semi[b]).wait()

    def start_scat(b):
        pltpu.async_copy(ones_v, cnt_sh.at[didx[b]], sems[b], add=True)

    def wait_scat(b):
        pltpu.make_async_copy(ones_v, cnt_sh.at[didx[b]], sems[b]).wait()

    @pl.when(n_i > 0)
    def _():
        start_fetch(0, 0)

    npair = (NCHUNKS // 16 + 2) // 2 + 1

    def body(i, carry):
        for b in (0, 1):
            k = i * 2 + b

            @pl.when(jnp.logical_and(k >= 1, k - 1 < n_i))
            def _():
                wait_scat(1 - b)

            @pl.when(k + 1 < n_i)
            def _():
                start_fetch(k + 1, 1 - b)

            @pl.when(k < n_i)
            def _():
                wait_fetch(b)
                start_scat(b)
        return carry

    lax.fori_loop(0, npair, body, 0)
    plsc.subcore_barrier()

    @pl.when(cid == 0)
    def _():
        def wout(c):
            pltpu.sync_copy(ind_sh.at[pl.ds(c * ZCH, ZCH)], ind_o.at[pl.ds(c * ZCH, ZCH)])
            pltpu.sync_copy(cnt_sh.at[pl.ds(c * ZCH, ZCH)], cntc_o.at[pl.ds(c * ZCH, ZCH)])

        _sub_chunk_loop(sid, NZCH, wout)

    @pl.when(cid == 1)
    def _():
        def wout(c):
            pltpu.sync_copy(cnt_sh.at[pl.ds(c * ZCH, ZCH)], cntv_o.at[pl.ds(c * ZCH, ZCH)])

        _sub_chunk_loop(sid, NZCH, wout)


def _gather_sc_body(hl, hr, src, dst, ga, gb,
                    sidx0, didx0, sidx1, didx1,
                    bufa0, bufb0, bufa1, bufb1,
                    semi0, semi1, semg0, semg1, semw0, semw1):
    cid = lax.axis_index("c")
    sid = lax.axis_index("s")
    wid = sid * 2 + cid
    n_i = NCHUNKS // 32 + jnp.where(wid < (NCHUNKS % 32), 1, 0)
    sidx = (sidx0, sidx1)
    didx = (didx0, didx1)
    bufa = (bufa0, bufa1)
    bufb = (bufb0, bufb1)
    semi = (semi0, semi1)
    semg = (semg0, semg1)
    semw = (semw0, semw1)

    def base(k):
        return (wid + k * 32) * ECH

    def start_idx(k, b):
        pltpu.async_copy(src.at[pl.ds(base(k), ECH)], sidx[b], semi[b])
        pltpu.async_copy(dst.at[pl.ds(base(k), ECH)], didx[b], semi[b])

    def wait_idx(b):
        pltpu.make_async_copy(src.at[pl.ds(0, ECH)], sidx[b], semi[b]).wait()
        pltpu.make_async_copy(dst.at[pl.ds(0, ECH)], didx[b], semi[b]).wait()

    def start_gather(b):
        pltpu.async_copy(hl.at[sidx[b]], bufa[b], semg[b])
        pltpu.async_copy(hr.at[didx[b]], bufb[b], semg[b])

    def wait_gather(b):
        pltpu.make_async_copy(hl.at[sidx[b]], bufa[b], semg[b]).wait()
        pltpu.make_async_copy(hr.at[didx[b]], bufb[b], semg[b]).wait()

    def start_wb(k, b):
        pltpu.async_copy(bufa[b], ga.at[pl.ds(base(k), ECH)], semw[b])
        pltpu.async_copy(bufb[b], gb.at[pl.ds(base(k), ECH)], semw[b])

    def wait_wb(b):
        pltpu.make_async_copy(bufa[b], ga.at[pl.ds(0, ECH)], semw[b]).wait()
        pltpu.make_async_copy(bufb[b], gb.at[pl.ds(0, ECH)], semw[b]).wait()

    @pl.when(n_i > 0)
    def _():
        start_idx(0, 0)

    npair = (NCHUNKS // 32 + 2) // 2 + 1

    def body(i, carry):
        for b in (0, 1):
            k = i * 2 + b

            @pl.when(jnp.logical_and(k >= 1, k - 1 < n_i))
            def _():
                wait_gather(1 - b)
                start_wb(k - 1, 1 - b)

            @pl.when(k + 1 < n_i)
            def _():
                start_idx(k + 1, 1 - b)

            @pl.when(k < n_i)
            def _():
                @pl.when(k >= 2)
                def _():
                    wait_wb(b)

                wait_idx(b)
                start_gather(b)
        return carry

    lax.fori_loop(0, npair, body, 0)

    par = n_i % 2
    @pl.when(jnp.logical_and(n_i >= 1, par == 0))
    def _():
        wait_wb(1)

    @pl.when(jnp.logical_and(n_i >= 2, par == 0))
    def _():
        wait_wb(0)

    @pl.when(jnp.logical_and(n_i >= 1, par == 1))
    def _():
        wait_wb(0)

    @pl.when(jnp.logical_and(n_i >= 2, par == 1))
    def _():
        wait_wb(1)


def _scatter_sc_body(t0, t1, dst, zeros32, sums0, sums1,
                     didx0, didx1, rows0, rows1,
                     semi0, semi1, sems0, sems1, tab_sh):
    cid = lax.axis_index("c")
    sid = lax.axis_index("s")

    def zero(c):
        pltpu.sync_copy(zeros32.at[pl.ds(c * ZCH, ZCH)], tab_sh.at[pl.ds(c * ZCH, ZCH)])

    _sub_chunk_loop(sid, NZCH, zero)
    plsc.subcore_barrier()

    didx = (didx0, didx1)
    rows = (rows0, rows1)
    semi = (semi0, semi1)
    sems = (sems0, sems1)
    n_i = NCHUNKS // 16 + jnp.where(sid < (NCHUNKS % 16), 1, 0)

    def base(k):
        return (sid + k * 16) * ECH

    def start_fetch(k, b):
        pltpu.async_copy(dst.at[pl.ds(base(k), ECH)], didx[b], semi[b])

        @pl.when(cid == 0)
        def _():
            pltpu.async_copy(t0.at[pl.ds(base(k), ECH)], rows[b], semi[b])

        @pl.when(cid == 1)
        def _():
            pltpu.async_copy(t1.at[pl.ds(base(k), ECH)], rows[b], semi[b])

    def wait_fetch(b):
        pltpu.make_async_copy(dst.at[pl.ds(0, ECH)], didx[b], semi[b]).wait()
        pltpu.make_async_copy(t0.at[pl.ds(0, ECH)], rows[b], semi[b]).wait()

    def start_scat(b):
        pltpu.async_copy(rows[b], tab_sh.at[didx[b]], sems[b], add=True)

    def wait_scat(b):
        pltpu.make_async_copy(rows[b], tab_sh.at[didx[b]], sems[b]).wait()

    @pl.when(n_i > 0)
    def _():
        start_fetch(0, 0)

    npair = (NCHUNKS // 16 + 2) // 2 + 1

    def body(i, carry):
        for b in (0, 1):
            k = i * 2 + b

            @pl.when(jnp.logical_and(k >= 1, k - 1 < n_i))
            def _():
                wait_scat(1 - b)

            @pl.when(k + 1 < n_i)
            def _():
                start_fetch(k + 1, 1 - b)

            @pl.when(k < n_i)
            def _():
                wait_fetch(b)
                start_scat(b)
        return carry

    lax.fori_loop(0, npair, body, 0)
    plsc.subcore_barrier()

    @pl.when(cid == 0)
    def _():
        def wout(c):
            pltpu.sync_copy(tab_sh.at[pl.ds(c * ZCH, ZCH)], sums0.at[pl.ds(c * ZCH, ZCH)])

        _sub_chunk_loop(sid, NZCH, wout)

    @pl.when(cid == 1)
    def _():
        def wout(c):
            pltpu.sync_copy(tab_sh.at[pl.ds(c * ZCH, ZCH)], sums1.at[pl.ds(c * ZCH, ZCH)])

        _sub_chunk_loop(sid, NZCH, wout)


def _cand_gather_sc_body(table, cand, out, idx_v, rows_v, sem):
    cid = lax.axis_index("c")
    sid = lax.axis_index("s")
    wid = sid * 2 + cid

    def body(i, carry):
        base = (wid + i * 32) * CCH
        pltpu.sync_copy(cand.at[pl.ds(base, CCH)], idx_v)
        pltpu.async_copy(table.at[idx_v], rows_v, sem).wait()
        pltpu.sync_copy(rows_v, out.at[pl.ds(base, CCH)])
        return carry

    n_i = NCCH // 32 + jnp.where(wid < (NCCH % 32), 1, 0)
    lax.fori_loop(0, n_i, body, 0)


@functools.cache
def _sc_kernels():
    mesh = plsc.VectorSubcoreMesh(core_axis_name="c", subcore_axis_name="s",
                                  num_cores=2, num_subcores=16)
    params = pltpu.CompilerParams(use_tc_tiling_on_sc=False)
    stats = pl.kernel(
        _stats_sc_body,
        out_type=(
            jax.ShapeDtypeStruct((NN, 8), f32),
            jax.ShapeDtypeStruct((NN, 8), f32),
            jax.ShapeDtypeStruct((NN, 8), f32),
        ),
        mesh=mesh,
        compiler_params=params,
        scratch_types=[
            pltpu.VMEM((CCH,), jnp.int32),
            pltpu.VMEM((ECH,), jnp.int32),
            pltpu.VMEM((ECH,), jnp.int32),
            pltpu.VMEM((ECH, 8), f32),
            pltpu.SemaphoreType.DMA,
            pltpu.SemaphoreType.DMA,
            pltpu.SemaphoreType.DMA,
            pltpu.SemaphoreType.DMA,
            pltpu.VMEM_SHARED((NN, 8), f32),
            pltpu.VMEM_SHARED((NN, 8), f32),
        ],
    )
    gather = pl.kernel(
        _gather_sc_body,
        out_type=(
            jax.ShapeDtypeStruct((NE, EMB), f32),
            jax.ShapeDtypeStruct((NE, EMB), f32),
        ),
        mesh=mesh,
        compiler_params=params,
        scratch_types=[
            pltpu.VMEM((ECH,), jnp.int32),
            pltpu.VMEM((ECH,), jnp.int32),
            pltpu.VMEM((ECH,), jnp.int32),
            pltpu.VMEM((ECH,), jnp.int32),
            pltpu.VMEM((ECH, EMB), f32),
            pltpu.VMEM((ECH, EMB), f32),
            pltpu.VMEM((ECH, EMB), f32),
            pltpu.VMEM((ECH, EMB), f32),
            pltpu.SemaphoreType.DMA,
            pltpu.SemaphoreType.DMA,
            pltpu.SemaphoreType.DMA,
            pltpu.SemaphoreType.DMA,
            pltpu.SemaphoreType.DMA,
            pltpu.SemaphoreType.DMA,
        ],
    )
    scatter = pl.kernel(
        _scatter_sc_body,
        out_type=(
            jax.ShapeDtypeStruct((NN, HALF), f32),
            jax.ShapeDtypeStruct((NN, HALF), f32),
        ),
        mesh=mesh,
        compiler_params=params,
        scratch_types=[
            pltpu.VMEM((ECH,), jnp.int32),
            pltpu.VMEM((ECH,), jnp.int32),
            pltpu.VMEM((ECH, HALF), f32),
            pltpu.VMEM((ECH, HALF), f32),
            pltpu.SemaphoreType.DMA,
            pltpu.SemaphoreType.DMA,
            pltpu.SemaphoreType.DMA,
            pltpu.SemaphoreType.DMA,
            pltpu.VMEM_SHARED((NN, HALF), f32),
        ],
    )
    cand_gather = pl.kernel(
        _cand_gather_sc_body,
        out_type=jax.ShapeDtypeStruct((NCAND, 8), f32),
        mesh=mesh,
        compiler_params=params,
        scratch_types=[
            pltpu.VMEM((CCH,), jnp.int32),
            pltpu.VMEM((CCH, 8), f32),
            pltpu.SemaphoreType.DMA,
        ],
    )
    return stats, gather, scatter, cand_gather


def _ln(x, g, b):
    m = jnp.mean(x, axis=-1, keepdims=True)
    v = jnp.mean((x - m) ** 2, axis=-1, keepdims=True)
    return (x - m) / jnp.sqrt(v + 1e-5) * g + b


def _lrelu(x):
    return jnp.where(x > 0, x, 0.01 * x)


def _mm(x, w):
    return lax.dot_general(x, w, (((1,), (1,)), ((), ())),
                           preferred_element_type=f32)


RB = 2000
RBE = 2000


def _row_spec(width):
    return pl.BlockSpec((RB, width), lambda i: (i, 0))


def _full_spec(shape):
    nd = len(shape)
    return pl.BlockSpec(shape, lambda i: (0,) * nd)


def _bc8(v):
    return jnp.broadcast_to(v.reshape(1, -1), (8, v.shape[-1]))


def _embed_body(rf, vf, ind,
                g0c, b0c, w1c, b1c, w2c, b2c,
                g0v, b0v, w1v, b1v, w2v, b2v,
                pe0, ped, wl, bl, wr,
                cons_o, var_o, hl_o, hr_o):
    h = _ln(rf[...], g0c[...][0:1, :], b0c[...][0:1, :])
    h = _lrelu(_mm(h, w1c[...]) + b1c[...][0:1, :])
    h = _lrelu(_mm(h, w2c[...]) + b2c[...][0:1, :])
    cons_o[...] = h
    hr_o[...] = _mm(h, wr[...])

    v = _ln(vf[...], g0v[...][0:1, :], b0v[...][0:1, :])
    v = _lrelu(_mm(v, w1v[...]) + b1v[...][0:1, :])
    v = _lrelu(_mm(v, w2v[...]) + b2v[...][0:1, :])
    v = v + pe0[...][0:1, :] + ind[...][:, 0:1] * ped[...][0:1, :]
    var_o[...] = v
    hl_o[...] = _mm(v, wl[...]) + bl[...][0:1, :]


def _embed_tc(rf, vf, ind8, ce, ve, pe0, ped, wl, bl, wr):
    outs = tuple(jax.ShapeDtypeStruct((NN, EMB), f32) for _ in range(4))
    w_specs = [
        _full_spec((8, 5)), _full_spec((8, 5)),
        _full_spec((EMB, 5)), _full_spec((8, EMB)),
        _full_spec((EMB, EMB)), _full_spec((8, EMB)),
        _full_spec((8, 19)), _full_spec((8, 19)),
        _full_spec((EMB, 19)), _full_spec((8, EMB)),
        _full_spec((EMB, EMB)), _full_spec((8, EMB)),
        _full_spec((8, EMB)), _full_spec((8, EMB)),
        _full_spec((EMB, EMB)), _full_spec((8, EMB)),
        _full_spec((EMB, EMB)),
    ]
    return pl.pallas_call(
        _embed_body,
        grid=(NN // RB,),
        in_specs=[_row_spec(5), _row_spec(19), _row_spec(8)] + w_specs,
        out_specs=tuple(_row_spec(EMB) for _ in range(4)),
        out_shape=outs,
    )(rf, vf, ind8,
      _bc8(ce['g0']), _bc8(ce['b0']), ce['W1'], _bc8(ce['b1']), ce['W2'], _bc8(ce['b2']),
      _bc8(ve['g0']), _bc8(ve['b0']), ve['W1'], _bc8(ve['b1']), ve['W2'], _bc8(ve['b2']),
      pe0, ped, wl, _bc8(bl), wr)


def _edge_body(ga, gb, g, b, wf, bf, t0_o, t1_o):
    t = _lrelu(_ln(ga[...] + gb[...], g[...][0:1, :], b[...][0:1, :]))
    msg = _mm(t, wf[...]) + bf[...][0:1, :]
    t0_o[...] = msg[:, :HALF]
    t1_o[...] = msg[:, HALF:]


def _edge_tc(ga, gb, g, b, wf, bf):
    spec64 = pl.BlockSpec((RBE, EMB), lambda i: (i, 0))
    spec32 = pl.BlockSpec((RBE, HALF), lambda i: (i, 0))
    return pl.pallas_call(
        _edge_body,
        grid=(NE // RBE,),
        in_specs=[spec64, spec64, _full_spec((8, EMB)), _full_spec((8, EMB)),
                  _full_spec((EMB, EMB)), _full_spec((8, EMB))],
        out_specs=(spec32, spec32),
        out_shape=(jax.ShapeDtypeStruct((NE, HALF), f32),
                   jax.ShapeDtypeStruct((NE, HALF), f32)),
    )(ga, gb, _bc8(g), _bc8(b), wf, _bc8(bf))


def _post1_body(s0, s1, cnt, cons, var,
                g, b, wo1, bo1, wo2, bo2, wl2, bl2, wr2,
                hl2_o, hr2_o):
    sums = jnp.concatenate([s0[...], s1[...]], axis=-1)
    c = cnt[...][:, 0:1]
    agg = sums / jnp.maximum(c, 1.0)
    h = _ln(agg, g[...][0:1, :], b[...][0:1, :])
    hcat = jnp.concatenate([h, cons[...]], axis=-1)
    ho = _lrelu(_mm(hcat, wo1[...]) + bo1[...][0:1, :])
    consn = _mm(ho, wo2[...]) + bo2[...][0:1, :]
    hl2_o[...] = _mm(consn, wl2[...]) + bl2[...][0:1, :]
    hr2_o[...] = _mm(var[...], wr2[...])


def _post1_tc(s0, s1, cnt, cons, var, p1, wl2, bl2, wr2):
    w_specs = [
        _full_spec((8, EMB)), _full_spec((8, EMB)),
        _full_spec((EMB, 2 * EMB)), _full_spec((8, EMB)),
        _full_spec((EMB, EMB)), _full_spec((8, EMB)),
        _full_spec((EMB, EMB)), _full_spec((8, EMB)),
        _full_spec((EMB, EMB)),
    ]
    return pl.pallas_call(
        _post1_body,
        grid=(NN // RB,),
        in_specs=[_row_spec(HALF), _row_spec(HALF), _row_spec(8),
                  _row_spec(EMB), _row_spec(EMB)] + w_specs,
        out_specs=(_row_spec(EMB), _row_spec(EMB)),
        out_shape=(jax.ShapeDtypeStruct((NN, EMB), f32),
                   jax.ShapeDtypeStruct((NN, EMB), f32)),
    )(s0, s1, cnt, cons, var,
      _bc8(p1['ln_pc_g']), _bc8(p1['ln_pc_b']),
      p1['Wo1'], _bc8(p1['bo1']), p1['Wo2'], _bc8(p1['bo2']),
      wl2, _bc8(bl2), wr2)


def _post2_body(s0, s1, cnt, var,
                g, b, wo1, bo1, wo2, bo2, w1h, b1h, w2h, b2h,
                out_o):
    sums = jnp.concatenate([s0[...], s1[...]], axis=-1)
    c = cnt[...][:, 0:1]
    agg = sums / jnp.maximum(c, 1.0)
    h = _ln(agg, g[...][0:1, :], b[...][0:1, :])
    hcat = jnp.concatenate([h, var[...]], axis=-1)
    ho = _lrelu(_mm(hcat, wo1[...]) + bo1[...][0:1, :])
    varn = _mm(ho, wo2[...]) + bo2[...][0:1, :]
    hh = _lrelu(_mm(varn, w1h[...]) + b1h[...][0:1, :])
    out_o[...] = _mm(hh, w2h[...]) + b2h[...][0:1, :]


def _post2_tc(s0, s1, cnt, var, p2, head):
    w_specs = [
        _full_spec((8, EMB)), _full_spec((8, EMB)),
        _full_spec((EMB, 2 * EMB)), _full_spec((8, EMB)),
        _full_spec((EMB, EMB)), _full_spec((8, EMB)),
        _full_spec((EMB, EMB)), _full_spec((8, EMB)),
        _full_spec((8, EMB)), _full_spec((8, 8)),
    ]
    b2h = jnp.broadcast_to(head['b2'].reshape(1, 1), (8, 8))
    return pl.pallas_call(
        _post2_body,
        grid=(NN // RB,),
        in_specs=[_row_spec(HALF), _row_spec(HALF), _row_spec(8),
                  _row_spec(EMB)] + w_specs,
        out_specs=_row_spec(8),
        out_shape=jax.ShapeDtypeStruct((NN, 8), f32),
    )(s0, s1, cnt, var,
      _bc8(p2['ln_pc_g']), _bc8(p2['ln_pc_b']),
      p2['Wo1'], _bc8(p2['bo1']), p2['Wo2'], _bc8(p2['bo2']),
      head['W1'], _bc8(head['b1']), _bc8(head['W2'][0]), b2h)


def kernel(row_features, edge_index, edge_attr, variable_features, candidates, params):
    del edge_attr
    e0 = edge_index[0].astype(jnp.int32)
    e1 = edge_index[1].astype(jnp.int32)
    cand = candidates.astype(jnp.int32)

    zeros32 = jnp.zeros((NN, HALF), f32)
    zeros8 = jnp.zeros((NN, 8), f32)
    ones8 = jnp.ones((ECH, 8), f32)

    pe = params['pos_emb']
    pe0 = _bc8(pe[0])
    ped = _bc8(pe[1] - pe[0])

    p1 = params['conv_vc']
    p2 = params['conv_cv']

    _stats_sc, _gather_sc, _scatter_sc, _cand_gather_sc = _sc_kernels()

    ind8, cnt_c, cnt_v = _stats_sc(cand, e0, e1, zeros8, ones8)
    cons, var, hl1, hr1 = _embed_tc(
        row_features, variable_features, ind8,
        params['cons_emb'], params['var_emb'], pe0, ped,
        p1['Wl'], p1['bl'], p1['Wr'])

    ga, gb = _gather_sc(hl1, hr1, e1, e0)
    t0, t1 = _edge_tc(ga, gb, p1['ln_f_g'], p1['ln_f_b'], p1['Wf'], p1['bf'])
    s0, s1 = _scatter_sc(t0, t1, e0, zeros32)
    hl2, hr2 = _post1_tc(s0, s1, cnt_c, cons, var, p1,
                         p2['Wl'], p2['bl'], p2['Wr'])

    ga2, gb2 = _gather_sc(hl2, hr2, e0, e1)
    t0b, t1b = _edge_tc(ga2, gb2, p2['ln_f_g'], p2['ln_f_b'], p2['Wf'], p2['bf'])
    s0b, s1b = _scatter_sc(t0b, t1b, e1, zeros32)
    out2d = _post2_tc(s0b, s1b, cnt_v, var, p2, params['head'])

    res2d = _cand_gather_sc(out2d, cand)
    return res2d[:, 0]
